# Initial kernel scaffold; baseline (speedup 1.0000x reference)
import jax
import jax.numpy as jnp
from jax.experimental import pallas as pl


def kernel(x, edge_index, edge_attr, g0_Wl, g0_Wr, g0_We, g0_att, g0_b, g1_Wl, g1_Wr, g1_We, g1_att, g1_b, g2_Wl, g2_Wr, g2_We, g2_att, g2_b, Wih, Whh, bih, bhh, fc1_W, fc1_b, fc2_W, fc2_b):
    raise NotImplementedError("write your pallas kernel here")


# speedup vs baseline: 1.0033x; 1.0033x over previous
import functools
import jax
import jax.numpy as jnp
from jax.experimental import pallas as pl
from jax.experimental.pallas import tpu as pltpu

T, N, E, F = 16, 2000, 32000, 128
H, CH, CT, LH = 4, 32, 4, 256
IN_LSTM = CT * H * N

KB = 1280


def _zx_kernel(emb_ref, wih_ref, out_ref):
    k = pl.program_id(0)

    @pl.when(k == 0)
    def _():
        out_ref[...] = jnp.zeros_like(out_ref)

    out_ref[...] += jax.lax.dot_general(
        emb_ref[...], wih_ref[...], (((1,), (1,)), ((), ())),
        preferred_element_type=jnp.float32)


def _lstm_head_kernel(zx_ref, whh_ref, b_ref, fc1w_ref, fc1b_ref,
                      fc2w_ref, out_ref):
    b = b_ref[...]

    zx = zx_ref[...]
    h = jnp.zeros((1, LH), dtype=jnp.float32)
    c = jnp.zeros((1, LH), dtype=jnp.float32)
    for t in range(T):
        z = zx[t:t + 1, :] + b + jax.lax.dot_general(
            h, whh_ref[...], (((1,), (1,)), ((), ())),
            preferred_element_type=jnp.float32)
        i = z[:, 0 * LH:1 * LH]
        f = z[:, 1 * LH:2 * LH]
        g = z[:, 2 * LH:3 * LH]
        o = z[:, 3 * LH:4 * LH]
        c = jax.nn.sigmoid(f) * c + jax.nn.sigmoid(i) * jnp.tanh(g)
        h = jax.nn.sigmoid(o) * jnp.tanh(c)
    last = jax.nn.relu(h)
    h1 = jax.nn.relu(jax.lax.dot_general(
        last, fc1w_ref[...], (((1,), (1,)), ((), ())),
        preferred_element_type=jnp.float32) + fc1b_ref[...])
    s = jnp.sum(h1 * fc2w_ref[...], axis=1, keepdims=True)
    out_ref[...] = jnp.broadcast_to(s, (1, 128))


def _gatv2(x, src, dst, ea, Wl, Wr, We, att, b):
    Hh, C = att.shape
    xl = (x @ Wl).reshape(N, Hh, C)
    xr = (x @ Wr).reshape(N, Hh, C)
    ee = (ea @ We).reshape(-1, Hh, C)
    m = jax.nn.leaky_relu(xl[src] + xr[dst] + ee, 0.2)
    alpha = jnp.sum(m * att[None, :, :], axis=-1)
    amax = jax.ops.segment_max(alpha, dst, num_segments=N)
    amax = jnp.where(jnp.isfinite(amax), amax, 0.0)
    ex = jnp.exp(alpha - amax[dst])
    den = jax.ops.segment_sum(ex, dst, num_segments=N)
    w = ex / (den[dst] + 1e-16)
    out = jax.ops.segment_sum(xl[src] * w[:, :, None], dst, num_segments=N)
    return out.reshape(N, Hh * C) + b


def kernel(x, edge_index, edge_attr, g0_Wl, g0_Wr, g0_We, g0_att, g0_b,
           g1_Wl, g1_Wr, g1_We, g1_att, g1_b, g2_Wl, g2_Wr, g2_We, g2_att,
           g2_b, Wih, Whh, bih, bhh, fc1_W, fc1_b, fc2_W, fc2_b):
    layers = [(g0_Wl, g0_Wr, g0_We, g0_att, g0_b),
              (g1_Wl, g1_Wr, g1_We, g1_att, g1_b),
              (g2_Wl, g2_Wr, g2_We, g2_att, g2_b)]

    def timestep(x_t, ei_t, ea_t):
        src, dst = ei_t[0], ei_t[1]
        h = x_t
        for (Wl, Wr, We, att, b) in layers:
            h = jax.nn.relu(_gatv2(h, src, dst, ea_t, Wl, Wr, We, att, b))
        return h.reshape(-1)

    emb = jax.vmap(timestep)(x, edge_index, edge_attr)

    zx = pl.pallas_call(
        _zx_kernel,
        grid=(IN_LSTM // KB,),
        in_specs=[
            pl.BlockSpec((T, KB), lambda k: (0, k)),
            pl.BlockSpec((4 * LH, KB), lambda k: (0, k)),
        ],
        out_specs=pl.BlockSpec((T, 4 * LH), lambda k: (0, 0)),
        out_shape=jax.ShapeDtypeStruct((T, 4 * LH), jnp.float32),
    )(emb, Wih)

    b = (bih + bhh).reshape(1, 4 * LH)
    out = pl.pallas_call(
        _lstm_head_kernel,
        out_shape=jax.ShapeDtypeStruct((1, 128), jnp.float32),
    )(zx, Whh, b, fc1_W, fc1_b.reshape(1, 256), fc2_W)
    return out[:, :1] + fc2_b.reshape(1, 1)


# speedup vs baseline: 32.1235x; 32.0183x over previous
import functools
import jax
import jax.numpy as jnp
from jax import lax
from jax.experimental import pallas as pl
from jax.experimental.pallas import tpu as pltpu, tpu_sc as plsc

T, N, E, F = 16, 2000, 32000, 128
H_, CH, CT, LH = 4, 32, 4, 256
IN_LSTM = CT * H_ * N

NP = 2048
EPAD = 32768
EW = EPAD // 16
B = 64
NBLK = EW // B
NG = B // 16
RPT = NP // 16
TT = T // 2

KB = 1280

_mesh = plsc.VectorSubcoreMesh(core_axis_name="c", subcore_axis_name="s",
                               num_cores=2, num_subcores=16)


def _dg(v, idx):
    return lax.gather(
        v, idx[:, None],
        lax.GatherDimensionNumbers(offset_dims=(), collapsed_slice_dims=(0,),
                                   start_index_map=(0,)),
        (1,), mode=lax.GatherScatterMode.PROMISE_IN_BOUNDS)


@functools.partial(
    pl.kernel,
    out_type=jax.ShapeDtypeStruct((T, NP + NP // 8, 128), jnp.float32),
    mesh=_mesh,
    scratch_types=[
        pltpu.VMEM((NBLK, B), jnp.int32),
        pltpu.VMEM((NBLK, B), jnp.int32),
        pltpu.VMEM((NBLK, B), jnp.int32),
        pltpu.VMEM((NBLK, B), jnp.int32),
        pltpu.VMEM((NBLK * NG, 16), jnp.float32),
        pltpu.VMEM((NBLK * NG, 16), jnp.float32),
        pltpu.VMEM((NBLK * NG, 16), jnp.int32),
        pltpu.VMEM((B, 128), jnp.float32),
        pltpu.VMEM((B, 128), jnp.float32),
        pltpu.VMEM((B, 128), jnp.float32),
        pltpu.VMEM((B, 128), jnp.float32),
        pltpu.VMEM((32, 128), jnp.float32),
        pltpu.VMEM((4, 128), jnp.float32),
        pltpu.SemaphoreType.DMA,
        pltpu.SemaphoreType.DMA,
        pltpu.VMEM_SHARED((NP + NP // 8, 128), jnp.float32),
    ],
)
def _edge128(xl2, xr2, sidx_h, didxg_h, dnid_h, ddid_h, ea0_h, ea1_h,
             dm8_h, wvec, outp, sidxt, didxgt, dnidt, ddidt, ea0b, ea1b,
             dm8b, Lr, Rr, wln, wld, zb, wvb, sem0, sem1, sp):
    cid = lax.axis_index("c")
    sid = lax.axis_index("s")
    iota16 = lax.iota(jnp.int32, 16)
    iotaf = iota16.astype(jnp.float32)
    one16 = jnp.ones((16,), jnp.float32)
    z16 = jnp.zeros((16,), jnp.float32)
    cvecs = [jnp.full((16,), i, jnp.int32) for i in range(16)]
    lhot = [jnp.maximum(one16 - (iotaf - float(h)) * (iotaf - float(h)),
                        0.0) for h in range(4)]

    pltpu.sync_copy(wvec, wvb)
    we0 = [wvb[0, pl.ds(k * 16, 16)] for k in range(8)]
    we1 = [wvb[1, pl.ds(k * 16, 16)] for k in range(8)]
    att = [wvb[2, pl.ds(k * 16, 16)] for k in range(8)]

    def zrow(r, _):
        for k in range(8):
            zb[r, pl.ds(k * 16, 16)] = z16
        return 0
    lax.fori_loop(0, 32, zrow, 0)

    def per_t(tt, _):
        tg = cid * TT + tt
        pltpu.sync_copy(sidx_h.at[tg, sid], sidxt)
        pltpu.sync_copy(didxg_h.at[tg, sid], didxgt)
        pltpu.sync_copy(dnid_h.at[tg, sid], dnidt)
        pltpu.sync_copy(ddid_h.at[tg, sid], ddidt)
        pltpu.sync_copy(ea0_h.at[tg, sid], ea0b)
        pltpu.sync_copy(ea1_h.at[tg, sid], ea1b)
        pltpu.sync_copy(dm8_h.at[tg, sid], dm8b)
        for j in range(4):
            pltpu.sync_copy(zb, sp.at[pl.ds(sid * RPT + j * 32, 32)])
        pltpu.sync_copy(zb.at[pl.ds(0, 16)],
                        sp.at[pl.ds(NP + sid * 16, 16)])
        plsc.subcore_barrier()

        def per_blk(b, _):
            cpL = pltpu.async_copy(xl2.at[sidxt.at[b]], Lr, sem0)
            cpR = pltpu.async_copy(xr2.at[didxgt.at[b]], Rr, sem1)
            cpL.wait()
            cpR.wait()

            def per_grp(g, _):
                gr = b * NG + g
                ea0v = ea0b[gr, pl.ds(0, 16)]
                ea1v = ea1b[gr, pl.ds(0, 16)]
                dmv = dm8b[gr, pl.ds(0, 16)]
                for i in range(16):
                    e = g * 16 + i
                    ea0e = _dg(ea0v, cvecs[i])
                    ea1e = _dg(ea1v, cvecs[i])
                    dme = _dg(dmv, cvecs[i]).astype(jnp.float32)
                    ls = []
                    ps = []
                    for k in range(8):
                        l = Lr[e, pl.ds(k * 16, 16)]
                        r = Rr[e, pl.ds(k * 16, 16)]
                        u = l + r + ea0e * we0[k] + ea1e * we1[k]
                        m = jnp.maximum(u, 0.2 * u)
                        ls.append(l)
                        ps.append(m * att[k])
                    exs = []
                    for h in range(4):
                        q = ps[2 * h] + ps[2 * h + 1]
                        for st in (1, 2, 4, 8):
                            q = q + _dg(q, iota16 ^ st)
                        exs.append(jnp.exp(q))
                    for k in range(8):
                        wln[e, pl.ds(k * 16, 16)] = ls[k] * exs[k // 2]
                    dv = (exs[0] * lhot[0] + exs[1] * lhot[1] +
                          exs[2] * lhot[2] + exs[3] * lhot[3])
                    for k in range(8):
                        dk = dme - float(k)
                        wld[e, pl.ds(k * 16, 16)] = dv * jnp.maximum(
                            one16 - dk * dk, 0.0)
                return 0

            lax.fori_loop(0, NG, per_grp, 0)

            for c in range(NG):
                pltpu.sync_copy(wln.at[pl.ds(c * 16, 16)],
                                sp.at[dnidt.at[b, pl.ds(c * 16, 16)]],
                                add=True)
                pltpu.sync_copy(wld.at[pl.ds(c * 16, 16)],
                                sp.at[ddidt.at[b, pl.ds(c * 16, 16)]],
                                add=True)
            return 0

        lax.fori_loop(0, NBLK, per_blk, 0)
        plsc.subcore_barrier()
        pltpu.sync_copy(sp.at[pl.ds(sid * RPT, RPT)],
                        outp.at[tg, pl.ds(sid * RPT, RPT)])
        pltpu.sync_copy(sp.at[pl.ds(NP + sid * 16, 16)],
                        outp.at[tg, pl.ds(NP + sid * 16, 16)])
        return 0

    lax.fori_loop(0, TT, per_t, 0)


@functools.partial(
    pl.kernel,
    out_type=jax.ShapeDtypeStruct((T, NP, 128), jnp.float32),
    mesh=_mesh,
    scratch_types=[
        pltpu.VMEM((NBLK, B), jnp.int32),
        pltpu.VMEM((NBLK, B), jnp.int32),
        pltpu.VMEM((NBLK, B), jnp.int32),
        pltpu.VMEM((NBLK * NG, 16), jnp.float32),
        pltpu.VMEM((NBLK * NG, 16), jnp.float32),
        pltpu.VMEM((B, 128), jnp.float32),
        pltpu.VMEM((B, 128), jnp.float32),
        pltpu.VMEM((B, 128), jnp.float32),
        pltpu.VMEM((32, 128), jnp.float32),
        pltpu.VMEM((4, 128), jnp.float32),
        pltpu.SemaphoreType.DMA,
        pltpu.SemaphoreType.DMA,
        pltpu.VMEM_SHARED((NP, 128), jnp.float32),
    ],
)
def _edge16(xl2, xr2, sidx_h, didxg_h, dnid_h, ea0_h, ea1_h, wvec, outp,
            sidxt, didxgt, dnidt, ea0b, ea1b, Lr, Rr, wl, zb, wvb, sem0,
            sem1, sp):
    cid = lax.axis_index("c")
    sid = lax.axis_index("s")
    iota16 = lax.iota(jnp.int32, 16)
    iotaf = iota16.astype(jnp.float32)
    one16 = jnp.ones((16,), jnp.float32)
    z16 = jnp.zeros((16,), jnp.float32)
    cvecs = [jnp.full((16,), i, jnp.int32) for i in range(16)]
    dsel = (iota16 * 4) & 15
    m4 = jnp.maximum(jnp.minimum(4.0 - iotaf, 1.0), 0.0)

    pltpu.sync_copy(wvec, wvb)
    we0 = wvb[0, pl.ds(0, 16)]
    we1 = wvb[1, pl.ds(0, 16)]
    att = wvb[2, pl.ds(0, 16)]

    def zrow(r, _):
        for k in range(8):
            zb[r, pl.ds(k * 16, 16)] = z16
        return 0
    lax.fori_loop(0, 32, zrow, 0)

    def zwl(r, _):
        for k in range(8):
            wl[r, pl.ds(k * 16, 16)] = z16
        return 0
    lax.fori_loop(0, B, zwl, 0)

    def per_t(tt, _):
        tg = cid * TT + tt
        pltpu.sync_copy(sidx_h.at[tg, sid], sidxt)
        pltpu.sync_copy(didxg_h.at[tg, sid], didxgt)
        pltpu.sync_copy(dnid_h.at[tg, sid], dnidt)
        pltpu.sync_copy(ea0_h.at[tg, sid], ea0b)
        pltpu.sync_copy(ea1_h.at[tg, sid], ea1b)
        for j in range(4):
            pltpu.sync_copy(zb, sp.at[pl.ds(sid * RPT + j * 32, 32)])
        plsc.subcore_barrier()

        def per_blk(b, _):
            cpL = pltpu.async_copy(xl2.at[sidxt.at[b]], Lr, sem0)
            cpR = pltpu.async_copy(xr2.at[didxgt.at[b]], Rr, sem1)
            cpL.wait()
            cpR.wait()

            def per_grp(g, _):
                gr = b * NG + g
                ea0v = ea0b[gr, pl.ds(0, 16)]
                ea1v = ea1b[gr, pl.ds(0, 16)]
                for i in range(16):
                    e = g * 16 + i
                    ea0e = _dg(ea0v, cvecs[i])
                    ea1e = _dg(ea1v, cvecs[i])
                    l = Lr[e, pl.ds(0, 16)]
                    r = Rr[e, pl.ds(0, 16)]
                    u = l + r + ea0e * we0 + ea1e * we1
                    m = jnp.maximum(u, 0.2 * u)
                    p = m * att
                    p = p + _dg(p, iota16 ^ 1)
                    p = p + _dg(p, iota16 ^ 2)
                    ex = jnp.exp(p)
                    wl[e, pl.ds(0, 16)] = l * ex
                    wl[e, pl.ds(16, 16)] = _dg(ex, dsel) * m4
                return 0

            lax.fori_loop(0, NG, per_grp, 0)

            for c in range(NG):
                pltpu.sync_copy(wl.at[pl.ds(c * 16, 16)],
                                sp.at[dnidt.at[b, pl.ds(c * 16, 16)]],
                                add=True)
            return 0

        lax.fori_loop(0, NBLK, per_blk, 0)
        plsc.subcore_barrier()
        pltpu.sync_copy(sp.at[pl.ds(sid * RPT, RPT)],
                        outp.at[tg, pl.ds(sid * RPT, RPT)])
        return 0

    lax.fori_loop(0, TT, per_t, 0)


def _proj0_kernel(x_ref, wl_ref, wr_ref, xl_ref, xr_ref):
    xv = x_ref[0]
    xl_ref[0] = jnp.dot(xv, wl_ref[...], preferred_element_type=jnp.float32)
    xr_ref[0] = jnp.dot(xv, wr_ref[...], preferred_element_type=jnp.float32)


def _mergeproj_kernel(pn_ref, pd_ref, dsel_ref, b_ref, wl_ref, wr_ref,
                      xl_ref, xr_ref):
    den = jnp.dot(pd_ref[0], dsel_ref[...],
                  preferred_element_type=jnp.float32) + 1e-16
    hm = jax.nn.relu(pn_ref[0] / den + b_ref[...])
    xl_ref[0] = jnp.dot(hm, wl_ref[...], preferred_element_type=jnp.float32)
    xr_ref[0] = jnp.dot(hm, wr_ref[...], preferred_element_type=jnp.float32)


def _mergefinal_kernel(p_ref, dsel_ref, b_ref, h_ref):
    pv = p_ref[0]
    den = jnp.dot(pv, dsel_ref[...],
                  preferred_element_type=jnp.float32) + 1e-16
    h_ref[0] = jax.nn.relu(pv[:, :16] / den + b_ref[...])


def _zx_kernel(emb_ref, wih_ref, out_ref):
    k = pl.program_id(0)

    @pl.when(k == 0)
    def _():
        out_ref[...] = jnp.zeros_like(out_ref)

    out_ref[...] += lax.dot_general(
        emb_ref[...], wih_ref[...], (((1,), (1,)), ((), ())),
        preferred_element_type=jnp.float32)


def _lstm_head_kernel(zx_ref, whh_ref, b_ref, fc1w_ref, fc1b_ref,
                      fc2w_ref, out_ref):
    b = b_ref[...]
    zx = zx_ref[...]
    h = jnp.zeros((1, LH), dtype=jnp.float32)
    c = jnp.zeros((1, LH), dtype=jnp.float32)
    for t in range(T):
        z = zx[t:t + 1, :] + b + lax.dot_general(
            h, whh_ref[...], (((1,), (1,)), ((), ())),
            preferred_element_type=jnp.float32)
        i = z[:, 0 * LH:1 * LH]
        f = z[:, 1 * LH:2 * LH]
        g = z[:, 2 * LH:3 * LH]
        o = z[:, 3 * LH:4 * LH]
        c = jax.nn.sigmoid(f) * c + jax.nn.sigmoid(i) * jnp.tanh(g)
        h = jax.nn.sigmoid(o) * jnp.tanh(c)
    last = jax.nn.relu(h)
    h1 = jax.nn.relu(lax.dot_general(
        last, fc1w_ref[...], (((1,), (1,)), ((), ())),
        preferred_element_type=jnp.float32) + fc1b_ref[...])
    s = jnp.sum(h1 * fc2w_ref[...], axis=1, keepdims=True)
    out_ref[...] = jnp.broadcast_to(s, (1, 128))


def _proj0(x_pad, Wl, Wr):
    return pl.pallas_call(
        _proj0_kernel,
        grid=(T,),
        in_specs=[
            pl.BlockSpec((1, NP, F), lambda t: (t, 0, 0)),
            pl.BlockSpec((F, 128), lambda t: (0, 0)),
            pl.BlockSpec((F, 128), lambda t: (0, 0)),
        ],
        out_specs=[
            pl.BlockSpec((1, NP, 128), lambda t: (t, 0, 0)),
            pl.BlockSpec((1, NP, 128), lambda t: (t, 0, 0)),
        ],
        out_shape=[jax.ShapeDtypeStruct((T, NP, 128), jnp.float32),
                   jax.ShapeDtypeStruct((T, NP, 128), jnp.float32)],
    )(x_pad, Wl, Wr)


def _mergeproj(pn, pd, dsel, bvec, Wl, Wr):
    return pl.pallas_call(
        _mergeproj_kernel,
        grid=(T,),
        in_specs=[
            pl.BlockSpec((1, NP, 128), lambda t: (t, 0, 0)),
            pl.BlockSpec((1, NP, 16), lambda t: (t, 0, 0)),
            pl.BlockSpec((16, 128), lambda t: (0, 0)),
            pl.BlockSpec((1, 128), lambda t: (0, 0)),
            pl.BlockSpec((128, 128), lambda t: (0, 0)),
            pl.BlockSpec((128, 128), lambda t: (0, 0)),
        ],
        out_specs=[
            pl.BlockSpec((1, NP, 128), lambda t: (t, 0, 0)),
            pl.BlockSpec((1, NP, 128), lambda t: (t, 0, 0)),
        ],
        out_shape=[jax.ShapeDtypeStruct((T, NP, 128), jnp.float32),
                   jax.ShapeDtypeStruct((T, NP, 128), jnp.float32)],
    )(pn, pd, dsel, bvec, Wl, Wr)


def _mergefinal(p, dsel, bvec):
    return pl.pallas_call(
        _mergefinal_kernel,
        grid=(T,),
        in_specs=[
            pl.BlockSpec((1, NP, 128), lambda t: (t, 0, 0)),
            pl.BlockSpec((128, 16), lambda t: (0, 0)),
            pl.BlockSpec((1, 16), lambda t: (0, 0)),
        ],
        out_specs=pl.BlockSpec((1, NP, 16), lambda t: (t, 0, 0)),
        out_shape=jax.ShapeDtypeStruct((T, NP, 16), jnp.float32),
    )(p, dsel, bvec)


def kernel(x, edge_index, edge_attr, g0_Wl, g0_Wr, g0_We, g0_att, g0_b,
           g1_Wl, g1_Wr, g1_We, g1_att, g1_b, g2_Wl, g2_Wr, g2_We, g2_att,
           g2_b, Wih, Whh, bih, bhh, fc1_W, fc1_b, fc2_W, fc2_b):
    src = edge_index[:, 0, :].astype(jnp.int32)
    dst = edge_index[:, 1, :].astype(jnp.int32)
    pad = EPAD - E
    srcp = jnp.concatenate([src, jnp.full((T, pad), N, jnp.int32)], 1)
    dstp = jnp.concatenate([dst, jnp.full((T, pad), N, jnp.int32)], 1)
    tno = (jnp.arange(T, dtype=jnp.int32) * NP)[:, None]
    sidx_h = (srcp + tno).reshape(T, 16, NBLK, B)
    didxg_h = (dstp + tno).reshape(T, 16, NBLK, B)
    dnid_h = dstp.reshape(T, 16, NBLK, B)
    ddid_h = (NP + dstp // 8).reshape(T, 16, NBLK, B)
    dm8_h = (dstp % 8).reshape(T, 16, NBLK * NG, 16)
    ea0_h = jnp.concatenate(
        [edge_attr[:, :, 0], jnp.zeros((T, pad), jnp.float32)],
        1).reshape(T, 16, NBLK * NG, 16)
    ea1_h = jnp.concatenate(
        [edge_attr[:, :, 1], jnp.zeros((T, pad), jnp.float32)],
        1).reshape(T, 16, NBLK * NG, 16)

    def wpack(We, att):
        hc = We.shape[1]
        row = jnp.zeros((4, 128), jnp.float32)
        row = row.at[0, :hc].set(We[0])
        row = row.at[1, :hc].set(We[1])
        row = row.at[2, :hc].set(att.reshape(hc))
        return row

    wvec0 = wpack(g0_We, g0_att)
    wvec1 = wpack(g1_We, g1_att)
    wvec2 = wpack(g2_We, g2_att)

    dsel128 = jnp.concatenate(
        [jnp.repeat(jnp.eye(4, dtype=jnp.float32), CH, axis=1),
         jnp.zeros((12, 128), jnp.float32)], 0)
    dsel16 = jnp.concatenate(
        [jnp.zeros((16, 16), jnp.float32),
         jnp.repeat(jnp.eye(4, dtype=jnp.float32), CT, axis=1),
         jnp.zeros((108, 16), jnp.float32)], 0)

    w2l = jnp.pad(g2_Wl, ((0, 0), (0, 112)))
    w2r = jnp.pad(g2_Wr, ((0, 0), (0, 112)))

    x_pad = jnp.pad(x, ((0, 0), (0, NP - N), (0, 0)))

    xl0, xr0 = _proj0(x_pad, g0_Wl, g0_Wr)
    p0 = _edge128(xl0.reshape(T * NP, 128), xr0.reshape(T * NP, 128),
                  sidx_h, didxg_h, dnid_h, ddid_h, ea0_h, ea1_h, dm8_h,
                  wvec0)
    xl1, xr1 = _mergeproj(p0[:, :NP], p0[:, NP:].reshape(T, NP, 16),
                          dsel128, g0_b.reshape(1, 128), g1_Wl, g1_Wr)
    p1 = _edge128(xl1.reshape(T * NP, 128), xr1.reshape(T * NP, 128),
                  sidx_h, didxg_h, dnid_h, ddid_h, ea0_h, ea1_h, dm8_h,
                  wvec1)
    xl2_, xr2_ = _mergeproj(p1[:, :NP], p1[:, NP:].reshape(T, NP, 16),
                            dsel128, g1_b.reshape(1, 128), w2l, w2r)
    p2 = _edge16(xl2_.reshape(T * NP, 128), xr2_.reshape(T * NP, 128),
                 sidx_h, didxg_h, dnid_h, ea0_h, ea1_h, wvec2)
    h2 = _mergefinal(p2, dsel16, g2_b.reshape(1, 16))
    emb = h2[:, :N, :].reshape(T, IN_LSTM)

    zx = pl.pallas_call(
        _zx_kernel,
        grid=(IN_LSTM // KB,),
        in_specs=[
            pl.BlockSpec((T, KB), lambda k: (0, k)),
            pl.BlockSpec((4 * LH, KB), lambda k: (0, k)),
        ],
        out_specs=pl.BlockSpec((T, 4 * LH), lambda k: (0, 0)),
        out_shape=jax.ShapeDtypeStruct((T, 4 * LH), jnp.float32),
    )(emb, Wih)

    bsum = (bih + bhh).reshape(1, 4 * LH)
    out = pl.pallas_call(
        _lstm_head_kernel,
        out_shape=jax.ShapeDtypeStruct((1, 128), jnp.float32),
    )(zx, Whh, bsum, fc1_W, fc1_b.reshape(1, 256), fc2_W)
    return out[:, :1] + fc2_b.reshape(1, 1)


# speedup vs baseline: 50.1949x; 1.5626x over previous
import functools
import jax
import jax.numpy as jnp
from jax import lax
from jax.experimental import pallas as pl
from jax.experimental.pallas import tpu as pltpu, tpu_sc as plsc

T, N, E, F = 16, 2000, 32000, 128
H_, CH, CT, LH = 4, 32, 4, 256
IN_LSTM = CT * H_ * N

NP = 2048
EPAD = 32768
EW = EPAD // 16
B = 64
NBLK = EW // B
NG = B // 16
RPT = NP // 16
TT = T // 2

KB = 1280

_mesh = plsc.VectorSubcoreMesh(core_axis_name="c", subcore_axis_name="s",
                               num_cores=2, num_subcores=16)


def _dg(v, idx):
    return lax.gather(
        v, idx[:, None],
        lax.GatherDimensionNumbers(offset_dims=(), collapsed_slice_dims=(0,),
                                   start_index_map=(0,)),
        (1,), mode=lax.GatherScatterMode.PROMISE_IN_BOUNDS)


@functools.partial(
    pl.kernel,
    out_type=jax.ShapeDtypeStruct((T, NP + NP // 8, 128), jnp.float32),
    mesh=_mesh,
    scratch_types=[
        pltpu.VMEM((NBLK, B), jnp.int32),
        pltpu.VMEM((NBLK, B), jnp.int32),
        pltpu.VMEM((NBLK, B), jnp.int32),
        pltpu.VMEM((NBLK, B), jnp.int32),
        pltpu.VMEM((NBLK * NG, 48), jnp.float32),
        pltpu.VMEM((B, 128), jnp.float32),
        pltpu.VMEM((B, 128), jnp.float32),
        pltpu.VMEM((B, 128), jnp.float32),
        pltpu.VMEM((B, 128), jnp.float32),
        pltpu.VMEM((B, 128), jnp.float32),
        pltpu.VMEM((B, 128), jnp.float32),
        pltpu.VMEM((32, 128), jnp.float32),
        pltpu.VMEM((4, 128), jnp.float32),
        pltpu.SemaphoreType.DMA,
        pltpu.SemaphoreType.DMA,
        pltpu.SemaphoreType.DMA,
        pltpu.SemaphoreType.DMA,
        pltpu.VMEM_SHARED((NP + NP // 8, 128), jnp.float32),
    ],
)
def _edge128(xl2, xr2, sidx_h, didxg_h, dnid_h, ddid_h, ea_h, wvec,
             outp, sidxt, didxgt, dnidt, ddidt, eab, LrA, RrA, LrB, RrB,
             wln, wld, zb, wvb, sem0, sem1, sem2, sem3, sp):
    cid = lax.axis_index("c")
    sid = lax.axis_index("s")
    iota16 = lax.iota(jnp.int32, 16)
    iotaf = iota16.astype(jnp.float32)
    one16 = jnp.ones((16,), jnp.float32)
    z16 = jnp.zeros((16,), jnp.float32)
    cvecs = [jnp.full((16,), i, jnp.int32) for i in range(16)]
    lhot = [jnp.maximum(one16 - (iotaf - float(h)) * (iotaf - float(h)),
                        0.0) for h in range(4)]

    pltpu.sync_copy(wvec, wvb)
    we0 = [wvb[0, pl.ds(k * 16, 16)] for k in range(8)]
    we1 = [wvb[1, pl.ds(k * 16, 16)] for k in range(8)]
    att = [wvb[2, pl.ds(k * 16, 16)] for k in range(8)]

    def zrow(r, _):
        for k in range(8):
            zb[r, pl.ds(k * 16, 16)] = z16
        return 0
    lax.fori_loop(0, 32, zrow, 0)

    def per_t(tt, _):
        tg = cid * TT + tt
        pltpu.sync_copy(sidx_h.at[tg, sid], sidxt)
        pltpu.sync_copy(didxg_h.at[tg, sid], didxgt)
        pltpu.sync_copy(dnid_h.at[tg, sid], dnidt)
        pltpu.sync_copy(ddid_h.at[tg, sid], ddidt)
        pltpu.sync_copy(ea_h.at[tg, sid], eab)
        for j in range(4):
            pltpu.sync_copy(zb, sp.at[pl.ds(sid * RPT + j * 32, 32)])
        pltpu.sync_copy(zb.at[pl.ds(0, 16)],
                        sp.at[pl.ds(NP + sid * 16, 16)])
        plsc.subcore_barrier()

        def _proc(b, Lr, Rr):
            def per_grp(g, _):
                gr = b * NG + g
                ea0v = eab[gr, pl.ds(0, 16)]
                ea1v = eab[gr, pl.ds(16, 16)]
                dmv = eab[gr, pl.ds(32, 16)]
                for i in range(16):
                    e = g * 16 + i
                    ea0e = _dg(ea0v, cvecs[i])
                    ea1e = _dg(ea1v, cvecs[i])
                    dme = _dg(dmv, cvecs[i])
                    ls = []
                    ps = []
                    for k in range(8):
                        l = Lr[e, pl.ds(k * 16, 16)]
                        r = Rr[e, pl.ds(k * 16, 16)]
                        u = l + r + ea0e * we0[k] + ea1e * we1[k]
                        m = jnp.maximum(u, 0.2 * u)
                        ls.append(l)
                        ps.append(m * att[k])
                    exs = []
                    for h in range(4):
                        q = ps[2 * h] + ps[2 * h + 1]
                        for st in (1, 2, 4, 8):
                            q = q + _dg(q, iota16 ^ st)
                        exs.append(jnp.exp(q))
                    for k in range(8):
                        wln[e, pl.ds(k * 16, 16)] = ls[k] * exs[k // 2]
                    dv = (exs[0] * lhot[0] + exs[1] * lhot[1] +
                          exs[2] * lhot[2] + exs[3] * lhot[3])
                    for k in range(8):
                        dk = dme - float(k)
                        wld[e, pl.ds(k * 16, 16)] = dv * jnp.maximum(
                            one16 - dk * dk, 0.0)
                return 0

            lax.fori_loop(0, NG, per_grp, 0)

            for c in range(NG):
                pltpu.sync_copy(wln.at[pl.ds(c * 16, 16)],
                                sp.at[dnidt.at[b, pl.ds(c * 16, 16)]],
                                add=True)
                pltpu.sync_copy(wld.at[pl.ds(c * 16, 16)],
                                sp.at[ddidt.at[b, pl.ds(c * 16, 16)]],
                                add=True)

        def per_blk2(bb, _):
            b0 = 2 * bb
            b1 = b0 + 1
            cpLA = pltpu.async_copy(xl2.at[sidxt.at[b0]], LrA, sem0)
            cpRA = pltpu.async_copy(xr2.at[didxgt.at[b0]], RrA, sem1)
            cpLB = pltpu.async_copy(xl2.at[sidxt.at[b1]], LrB, sem2)
            cpRB = pltpu.async_copy(xr2.at[didxgt.at[b1]], RrB, sem3)
            cpLA.wait()
            cpRA.wait()
            _proc(b0, LrA, RrA)
            cpLB.wait()
            cpRB.wait()
            _proc(b1, LrB, RrB)
            return 0

        lax.fori_loop(0, NBLK // 2, per_blk2, 0)
        plsc.subcore_barrier()
        pltpu.sync_copy(sp.at[pl.ds(sid * RPT, RPT)],
                        outp.at[tg, pl.ds(sid * RPT, RPT)])
        pltpu.sync_copy(sp.at[pl.ds(NP + sid * 16, 16)],
                        outp.at[tg, pl.ds(NP + sid * 16, 16)])
        return 0

    lax.fori_loop(0, TT, per_t, 0)


@functools.partial(
    pl.kernel,
    out_type=jax.ShapeDtypeStruct((T, NP, 128), jnp.float32),
    mesh=_mesh,
    scratch_types=[
        pltpu.VMEM((NBLK, B), jnp.int32),
        pltpu.VMEM((NBLK, B), jnp.int32),
        pltpu.VMEM((NBLK, B), jnp.int32),
        pltpu.VMEM((NBLK * NG, 48), jnp.float32),
        pltpu.VMEM((B, 128), jnp.float32),
        pltpu.VMEM((B, 128), jnp.float32),
        pltpu.VMEM((B, 128), jnp.float32),
        pltpu.VMEM((32, 128), jnp.float32),
        pltpu.VMEM((4, 128), jnp.float32),
        pltpu.SemaphoreType.DMA,
        pltpu.SemaphoreType.DMA,
        pltpu.VMEM_SHARED((NP, 128), jnp.float32),
    ],
)
def _edge16(xl2, xr2, sidx_h, didxg_h, dnid_h, ea_h, wvec, outp,
            sidxt, didxgt, dnidt, eab, Lr, Rr, wl, zb, wvb, sem0,
            sem1, sp):
    cid = lax.axis_index("c")
    sid = lax.axis_index("s")
    iota16 = lax.iota(jnp.int32, 16)
    iotaf = iota16.astype(jnp.float32)
    one16 = jnp.ones((16,), jnp.float32)
    z16 = jnp.zeros((16,), jnp.float32)
    cvecs = [jnp.full((16,), i, jnp.int32) for i in range(16)]
    dsel = (iota16 * 4) & 15
    m4 = jnp.maximum(jnp.minimum(4.0 - iotaf, 1.0), 0.0)

    pltpu.sync_copy(wvec, wvb)
    we0 = wvb[0, pl.ds(0, 16)]
    we1 = wvb[1, pl.ds(0, 16)]
    att = wvb[2, pl.ds(0, 16)]

    def zrow(r, _):
        for k in range(8):
            zb[r, pl.ds(k * 16, 16)] = z16
        return 0
    lax.fori_loop(0, 32, zrow, 0)

    def zwl(r, _):
        for k in range(8):
            wl[r, pl.ds(k * 16, 16)] = z16
        return 0
    lax.fori_loop(0, B, zwl, 0)

    def per_t(tt, _):
        tg = cid * TT + tt
        pltpu.sync_copy(sidx_h.at[tg, sid], sidxt)
        pltpu.sync_copy(didxg_h.at[tg, sid], didxgt)
        pltpu.sync_copy(dnid_h.at[tg, sid], dnidt)
        pltpu.sync_copy(ea_h.at[tg, sid], eab)
        for j in range(4):
            pltpu.sync_copy(zb, sp.at[pl.ds(sid * RPT + j * 32, 32)])
        plsc.subcore_barrier()

        def per_blk(b, _):
            cpL = pltpu.async_copy(xl2.at[sidxt.at[b]], Lr, sem0)
            cpR = pltpu.async_copy(xr2.at[didxgt.at[b]], Rr, sem1)
            cpL.wait()
            cpR.wait()

            def per_grp(g, _):
                gr = b * NG + g
                ea0v = eab[gr, pl.ds(0, 16)]
                ea1v = eab[gr, pl.ds(16, 16)]
                for i in range(16):
                    e = g * 16 + i
                    ea0e = _dg(ea0v, cvecs[i])
                    ea1e = _dg(ea1v, cvecs[i])
                    l = Lr[e, pl.ds(0, 16)]
                    r = Rr[e, pl.ds(0, 16)]
                    u = l + r + ea0e * we0 + ea1e * we1
                    m = jnp.maximum(u, 0.2 * u)
                    p = m * att
                    p = p + _dg(p, iota16 ^ 1)
                    p = p + _dg(p, iota16 ^ 2)
                    ex = jnp.exp(p)
                    wl[e, pl.ds(0, 16)] = l * ex
                    wl[e, pl.ds(16, 16)] = _dg(ex, dsel) * m4
                return 0

            lax.fori_loop(0, NG, per_grp, 0)

            for c in range(NG):
                pltpu.sync_copy(wl.at[pl.ds(c * 16, 16)],
                                sp.at[dnidt.at[b, pl.ds(c * 16, 16)]],
                                add=True)
            return 0

        lax.fori_loop(0, NBLK, per_blk, 0)
        plsc.subcore_barrier()
        pltpu.sync_copy(sp.at[pl.ds(sid * RPT, RPT)],
                        outp.at[tg, pl.ds(sid * RPT, RPT)])
        return 0

    lax.fori_loop(0, TT, per_t, 0)


def _proj0_kernel(x_ref, wl_ref, wr_ref, xl_ref, xr_ref):
    xv = x_ref[0]
    xl_ref[0] = jnp.dot(xv, wl_ref[...], preferred_element_type=jnp.float32)
    xr_ref[0] = jnp.dot(xv, wr_ref[...], preferred_element_type=jnp.float32)


def _mergeproj_kernel(pn_ref, pd_ref, dsel_ref, b_ref, wl_ref, wr_ref,
                      xl_ref, xr_ref):
    den = jnp.dot(pd_ref[0], dsel_ref[...],
                  preferred_element_type=jnp.float32) + 1e-16
    hm = jax.nn.relu(pn_ref[0] / den + b_ref[...])
    xl_ref[0] = jnp.dot(hm, wl_ref[...], preferred_element_type=jnp.float32)
    xr_ref[0] = jnp.dot(hm, wr_ref[...], preferred_element_type=jnp.float32)


def _mergefinal_kernel(p_ref, dsel_ref, b_ref, h_ref):
    pv = p_ref[0]
    den = jnp.dot(pv, dsel_ref[...],
                  preferred_element_type=jnp.float32) + 1e-16
    h_ref[0] = jax.nn.relu(pv[:, :16] / den + b_ref[...])


def _zx_kernel(emb_ref, wih_ref, out_ref):
    k = pl.program_id(0)

    @pl.when(k == 0)
    def _():
        out_ref[...] = jnp.zeros_like(out_ref)

    out_ref[...] += lax.dot_general(
        emb_ref[...], wih_ref[...], (((1,), (1,)), ((), ())),
        preferred_element_type=jnp.float32)


def _lstm_head_kernel(zx_ref, whh_ref, b_ref, fc1w_ref, fc1b_ref,
                      fc2w_ref, out_ref):
    b = b_ref[...]
    zx = zx_ref[...]
    h = jnp.zeros((1, LH), dtype=jnp.float32)
    c = jnp.zeros((1, LH), dtype=jnp.float32)
    for t in range(T):
        z = zx[t:t + 1, :] + b + lax.dot_general(
            h, whh_ref[...], (((1,), (1,)), ((), ())),
            preferred_element_type=jnp.float32)
        i = z[:, 0 * LH:1 * LH]
        f = z[:, 1 * LH:2 * LH]
        g = z[:, 2 * LH:3 * LH]
        o = z[:, 3 * LH:4 * LH]
        c = jax.nn.sigmoid(f) * c + jax.nn.sigmoid(i) * jnp.tanh(g)
        h = jax.nn.sigmoid(o) * jnp.tanh(c)
    last = jax.nn.relu(h)
    h1 = jax.nn.relu(lax.dot_general(
        last, fc1w_ref[...], (((1,), (1,)), ((), ())),
        preferred_element_type=jnp.float32) + fc1b_ref[...])
    s = jnp.sum(h1 * fc2w_ref[...], axis=1, keepdims=True)
    out_ref[...] = jnp.broadcast_to(s, (1, 128))


def _proj0(x_pad, Wl, Wr):
    return pl.pallas_call(
        _proj0_kernel,
        grid=(T,),
        in_specs=[
            pl.BlockSpec((1, NP, F), lambda t: (t, 0, 0)),
            pl.BlockSpec((F, 128), lambda t: (0, 0)),
            pl.BlockSpec((F, 128), lambda t: (0, 0)),
        ],
        out_specs=[
            pl.BlockSpec((1, NP, 128), lambda t: (t, 0, 0)),
            pl.BlockSpec((1, NP, 128), lambda t: (t, 0, 0)),
        ],
        out_shape=[jax.ShapeDtypeStruct((T, NP, 128), jnp.float32),
                   jax.ShapeDtypeStruct((T, NP, 128), jnp.float32)],
    )(x_pad, Wl, Wr)


def _mergeproj(pn, pd, dsel, bvec, Wl, Wr):
    return pl.pallas_call(
        _mergeproj_kernel,
        grid=(T,),
        in_specs=[
            pl.BlockSpec((1, NP, 128), lambda t: (t, 0, 0)),
            pl.BlockSpec((1, NP, 16), lambda t: (t, 0, 0)),
            pl.BlockSpec((16, 128), lambda t: (0, 0)),
            pl.BlockSpec((1, 128), lambda t: (0, 0)),
            pl.BlockSpec((128, 128), lambda t: (0, 0)),
            pl.BlockSpec((128, 128), lambda t: (0, 0)),
        ],
        out_specs=[
            pl.BlockSpec((1, NP, 128), lambda t: (t, 0, 0)),
            pl.BlockSpec((1, NP, 128), lambda t: (t, 0, 0)),
        ],
        out_shape=[jax.ShapeDtypeStruct((T, NP, 128), jnp.float32),
                   jax.ShapeDtypeStruct((T, NP, 128), jnp.float32)],
    )(pn, pd, dsel, bvec, Wl, Wr)


def _mergefinal(p, dsel, bvec):
    return pl.pallas_call(
        _mergefinal_kernel,
        grid=(T,),
        in_specs=[
            pl.BlockSpec((1, NP, 128), lambda t: (t, 0, 0)),
            pl.BlockSpec((128, 16), lambda t: (0, 0)),
            pl.BlockSpec((1, 16), lambda t: (0, 0)),
        ],
        out_specs=pl.BlockSpec((1, NP, 16), lambda t: (t, 0, 0)),
        out_shape=jax.ShapeDtypeStruct((T, NP, 16), jnp.float32),
    )(p, dsel, bvec)


def kernel(x, edge_index, edge_attr, g0_Wl, g0_Wr, g0_We, g0_att, g0_b,
           g1_Wl, g1_Wr, g1_We, g1_att, g1_b, g2_Wl, g2_Wr, g2_We, g2_att,
           g2_b, Wih, Whh, bih, bhh, fc1_W, fc1_b, fc2_W, fc2_b):
    src = edge_index[:, 0, :].astype(jnp.int32)
    dst = edge_index[:, 1, :].astype(jnp.int32)
    pad = EPAD - E
    srcp = jnp.concatenate([src, jnp.full((T, pad), N, jnp.int32)], 1)
    dstp = jnp.concatenate([dst, jnp.full((T, pad), N, jnp.int32)], 1)
    tno = (jnp.arange(T, dtype=jnp.int32) * NP)[:, None]
    sidx_h = (srcp + tno).reshape(T, 16, NBLK, B)
    didxg_h = (dstp + tno).reshape(T, 16, NBLK, B)
    dnid_h = dstp.reshape(T, 16, NBLK, B)
    ddid_h = (NP + dstp // 8).reshape(T, 16, NBLK, B)
    ea0p = jnp.concatenate(
        [edge_attr[:, :, 0], jnp.zeros((T, pad), jnp.float32)],
        1).reshape(T, 16, NBLK * NG, 16)
    ea1p = jnp.concatenate(
        [edge_attr[:, :, 1], jnp.zeros((T, pad), jnp.float32)],
        1).reshape(T, 16, NBLK * NG, 16)
    dm8f = (dstp % 8).astype(jnp.float32).reshape(T, 16, NBLK * NG, 16)
    ea_h = jnp.concatenate([ea0p, ea1p, dm8f], axis=3)

    def wpack(We, att):
        hc = We.shape[1]
        row = jnp.zeros((4, 128), jnp.float32)
        row = row.at[0, :hc].set(We[0])
        row = row.at[1, :hc].set(We[1])
        row = row.at[2, :hc].set(att.reshape(hc))
        return row

    wvec0 = wpack(g0_We, g0_att)
    wvec1 = wpack(g1_We, g1_att)
    wvec2 = wpack(g2_We, g2_att)

    dsel128 = jnp.concatenate(
        [jnp.repeat(jnp.eye(4, dtype=jnp.float32), CH, axis=1),
         jnp.zeros((12, 128), jnp.float32)], 0)
    dsel16 = jnp.concatenate(
        [jnp.zeros((16, 16), jnp.float32),
         jnp.repeat(jnp.eye(4, dtype=jnp.float32), CT, axis=1),
         jnp.zeros((108, 16), jnp.float32)], 0)

    w2l = jnp.pad(g2_Wl, ((0, 0), (0, 112)))
    w2r = jnp.pad(g2_Wr, ((0, 0), (0, 112)))

    x_pad = jnp.pad(x, ((0, 0), (0, NP - N), (0, 0)))

    xl0, xr0 = _proj0(x_pad, g0_Wl, g0_Wr)
    p0 = _edge128(xl0.reshape(T * NP, 128), xr0.reshape(T * NP, 128),
                  sidx_h, didxg_h, dnid_h, ddid_h, ea_h,
                  wvec0)
    xl1, xr1 = _mergeproj(p0[:, :NP], p0[:, NP:].reshape(T, NP, 16),
                          dsel128, g0_b.reshape(1, 128), g1_Wl, g1_Wr)
    p1 = _edge128(xl1.reshape(T * NP, 128), xr1.reshape(T * NP, 128),
                  sidx_h, didxg_h, dnid_h, ddid_h, ea_h,
                  wvec1)
    xl2_, xr2_ = _mergeproj(p1[:, :NP], p1[:, NP:].reshape(T, NP, 16),
                            dsel128, g1_b.reshape(1, 128), w2l, w2r)
    p2 = _edge16(xl2_.reshape(T * NP, 128), xr2_.reshape(T * NP, 128),
                 sidx_h, didxg_h, dnid_h, ea_h, wvec2)
    h2 = _mergefinal(p2, dsel16, g2_b.reshape(1, 16))
    emb = h2[:, :N, :].reshape(T, IN_LSTM)

    zx = pl.pallas_call(
        _zx_kernel,
        grid=(IN_LSTM // KB,),
        in_specs=[
            pl.BlockSpec((T, KB), lambda k: (0, k)),
            pl.BlockSpec((4 * LH, KB), lambda k: (0, k)),
        ],
        out_specs=pl.BlockSpec((T, 4 * LH), lambda k: (0, 0)),
        out_shape=jax.ShapeDtypeStruct((T, 4 * LH), jnp.float32),
    )(emb, Wih)

    bsum = (bih + bhh).reshape(1, 4 * LH)
    out = pl.pallas_call(
        _lstm_head_kernel,
        out_shape=jax.ShapeDtypeStruct((1, 128), jnp.float32),
    )(zx, Whh, bsum, fc1_W, fc1_b.reshape(1, 256), fc2_W)
    return out[:, :1] + fc2_b.reshape(1, 1)
